# Initial kernel scaffold; baseline (speedup 1.0000x reference)
import jax
import jax.numpy as jnp
from jax.experimental import pallas as pl


def kernel(inputs, targets):
    raise NotImplementedError("write your pallas kernel here")


# speedup vs baseline: 1.1968x; 1.1968x over previous
import math

import jax
import jax.numpy as jnp
from jax.experimental import pallas as pl


def _body(x_ref, t_ref, o_ref):
    C = x_ref.shape[1]
    x = x_ref[0]
    tgt = t_ref[0]
    best = x[0]
    pred = jnp.zeros_like(tgt)
    for c in range(1, C):
        m = x[c] > best
        best = jnp.where(m, x[c], best)
        pred = jnp.where(m, c, pred)
    eq = pred == tgt
    one = jnp.ones_like(best)
    zero = jnp.zeros_like(best)
    rows = []
    for c in range(C):
        pc = pred == c
        tc = tgt == c
        rows.append(jnp.sum(jnp.where(pc & tc, one, zero), axis=0, keepdims=True))
        rows.append(jnp.sum(jnp.where(pc, one, zero), axis=0, keepdims=True))
        rows.append(jnp.sum(jnp.where(tc, one, zero), axis=0, keepdims=True))
    cnt = jnp.concatenate(rows, axis=0)
    i = pl.program_id(1)

    @pl.when(i == 0)
    def _init():
        o_ref[0] = cnt

    @pl.when(i > 0)
    def _acc():
        o_ref[0] = o_ref[0] + cnt


def kernel(inputs, targets):
    eps = 1e-05
    B, C, D, H, W = inputs.shape
    N = D * H * W
    L = math.gcd(N, 512)
    S = N // L
    R = math.gcd(S, 256)
    G = S // R
    x = inputs.reshape(B, C, S, L)
    t = targets.reshape(B, S, L).astype(jnp.int32)
    counts = pl.pallas_call(
        _body,
        grid=(B, G),
        in_specs=[
            pl.BlockSpec((1, C, R, L), lambda b, i: (b, 0, i, 0)),
            pl.BlockSpec((1, R, L), lambda b, i: (b, i, 0)),
        ],
        out_specs=pl.BlockSpec((1, 3 * C, L), lambda b, i: (b, 0, 0)),
        out_shape=jax.ShapeDtypeStruct((B, 3 * C, L), jnp.float32),
    )(x, t)
    cnt = counts.sum(axis=2).reshape(B, C, 3)
    tp, cp, ct = cnt[..., 0], cnt[..., 1], cnt[..., 2]
    loss = 2.0 * tp / (cp + ct + eps)
    return loss[:, 1:].mean(axis=1)


# speedup vs baseline: 1.1973x; 1.0004x over previous
import math

import jax
import jax.numpy as jnp
from jax.experimental import pallas as pl
from jax.experimental.pallas import tpu as pltpu


def _body(x_ref, t_ref, o_ref):
    C = x_ref.shape[1]
    x = x_ref[0]
    tgt = t_ref[0]
    best = x[0]
    pred = jnp.zeros_like(tgt)
    for c in range(1, C):
        m = x[c] > best
        best = jnp.where(m, x[c], best)
        pred = jnp.where(m, c, pred)
    eq = pred == tgt
    one = jnp.ones_like(best)
    zero = jnp.zeros_like(best)
    rows = []
    for c in range(C):
        pc = pred == c
        tc = tgt == c
        rows.append(jnp.sum(jnp.where(pc & tc, one, zero), axis=0, keepdims=True))
        rows.append(jnp.sum(jnp.where(pc, one, zero), axis=0, keepdims=True))
        rows.append(jnp.sum(jnp.where(tc, one, zero), axis=0, keepdims=True))
    cnt = jnp.concatenate(rows, axis=0)
    i = pl.program_id(1)

    @pl.when(i == 0)
    def _init():
        o_ref[0] = cnt

    @pl.when(i > 0)
    def _acc():
        o_ref[0] = o_ref[0] + cnt


def kernel(inputs, targets):
    eps = 1e-05
    B, C, D, H, W = inputs.shape
    N = D * H * W
    L = math.gcd(N, 512)
    S = N // L
    R = math.gcd(S, 256)
    G = S // R
    x = inputs.reshape(B, C, S, L)
    t = targets.reshape(B, S, L).astype(jnp.int32)
    counts = pl.pallas_call(
        _body,
        grid=(B, G),
        in_specs=[
            pl.BlockSpec((1, C, R, L), lambda b, i: (b, 0, i, 0)),
            pl.BlockSpec((1, R, L), lambda b, i: (b, i, 0)),
        ],
        out_specs=pl.BlockSpec((1, 3 * C, L), lambda b, i: (b, 0, 0)),
        out_shape=jax.ShapeDtypeStruct((B, 3 * C, L), jnp.float32),
        compiler_params=pltpu.CompilerParams(
            dimension_semantics=("parallel", "arbitrary")),
    )(x, t)
    cnt = counts.sum(axis=2).reshape(B, C, 3)
    tp, cp, ct = cnt[..., 0], cnt[..., 1], cnt[..., 2]
    loss = 2.0 * tp / (cp + ct + eps)
    return loss[:, 1:].mean(axis=1)
